# baseline (speedup 1.0000x reference)
import functools

import jax
import jax.numpy as jnp
from jax import lax
from jax.experimental import pallas as pl
from jax.experimental.pallas import tpu as pltpu
from jax.experimental.pallas import tpu_sc as plsc

DMODEL = 1024
VOCAB = 100000
BATCH = 4
SEQLEN = 2048
TOTAL = BATCH * SEQLEN
NUM_WORKERS = 32
L_PER_W = SEQLEN // NUM_WORKERS
CHUNK = 16
M_PER_B = L_PER_W // CHUNK
NVEC = DMODEL // 16


def _position_embedding():
    pos = jnp.arange(SEQLEN, dtype=jnp.float32)[:, None]
    i = jnp.arange(DMODEL, dtype=jnp.float32)[None, :]
    inv_freq = 1.0 / jnp.power(10000.0, 2.0 * i / DMODEL)
    ang = pos * inv_freq
    dim_idx = jnp.arange(DMODEL)
    pe = jnp.where((dim_idx % 2 == 0)[None, :], jnp.sin(ang), jnp.cos(ang))
    return pe.astype(jnp.float32)


@functools.partial(
    pl.kernel,
    out_type=jax.ShapeDtypeStruct((TOTAL, DMODEL), jnp.float32),
    mesh=plsc.VectorSubcoreMesh(core_axis_name="c", subcore_axis_name="s"),
    scratch_types=[
        pltpu.VMEM((BATCH * L_PER_W,), jnp.int32),
        pltpu.VMEM((L_PER_W, DMODEL), jnp.float32),
        pltpu.VMEM((CHUNK, DMODEL), jnp.float32),
        pltpu.VMEM((CHUNK, DMODEL), jnp.float32),
        pltpu.SemaphoreType.DMA,
        pltpu.SemaphoreType.DMA,
    ],
)
def _sc_embed(ids_hbm, pe_hbm, table_hbm, out_hbm, idx_v, pe_v, buf0, buf1,
              sem0, sem1):
    wid = lax.axis_index("s") * 2 + lax.axis_index("c")
    l0 = wid * L_PER_W

    pltpu.sync_copy(pe_hbm.at[pl.ds(l0, L_PER_W)], pe_v)
    for b in range(BATCH):
        pltpu.sync_copy(
            ids_hbm.at[pl.ds(b * SEQLEN + l0, L_PER_W)],
            idx_v.at[pl.ds(b * L_PER_W, L_PER_W)],
        )

    bufs = (buf0, buf1)
    sems = (sem0, sem1)

    def start(c):
        return pltpu.async_copy(
            table_hbm.at[idx_v.at[pl.ds(c * CHUNK, CHUNK)]],
            bufs[c % 2], sems[c % 2],
        )

    pending = start(0)
    for c in range(BATCH * M_PER_B):
        buf = bufs[c % 2]
        pending.wait()
        if c + 1 < BATCH * M_PER_B:
            pending = start(c + 1)
        m = c % M_PER_B
        b = c // M_PER_B

        def add_pe(k, _):
            r = k >> 6
            off = (k & 63) * 16
            vec = pe_v[m * CHUNK + r, pl.ds(off, 16)]
            plsc.addupdate(buf.at[r, pl.ds(off, 16)], vec)
            return _

        lax.fori_loop(0, CHUNK * NVEC, add_pe, 0)
        out_base = b * SEQLEN + l0 + m * CHUNK
        pltpu.sync_copy(buf, out_hbm.at[pl.ds(out_base, CHUNK)])


def kernel(seq, table):
    pe = _position_embedding()
    flat_ids = seq.reshape(TOTAL).astype(jnp.int32)
    out = _sc_embed(flat_ids, pe, table)
    return out.reshape(BATCH, SEQLEN, DMODEL)


# speedup vs baseline: 1.4959x; 1.4959x over previous
import functools

import jax
import jax.numpy as jnp
from jax import lax
from jax.experimental import pallas as pl
from jax.experimental.pallas import tpu as pltpu
from jax.experimental.pallas import tpu_sc as plsc

DMODEL = 1024
VOCAB = 100000
BATCH = 4
SEQLEN = 2048
TOTAL = BATCH * SEQLEN
NUM_WORKERS = 32
L_PER_W = SEQLEN // NUM_WORKERS
CHUNK = 16
M_PER_B = L_PER_W // CHUNK
NCHUNKS = BATCH * M_PER_B
NVEC = DMODEL // 16
NBUF = 3


def _position_embedding():
    pos = jnp.arange(SEQLEN, dtype=jnp.float32)[:, None]
    i = jnp.arange(DMODEL, dtype=jnp.float32)[None, :]
    inv_freq = 1.0 / jnp.power(10000.0, 2.0 * i / DMODEL)
    ang = pos * inv_freq
    dim_idx = jnp.arange(DMODEL)
    pe = jnp.where((dim_idx % 2 == 0)[None, :], jnp.sin(ang), jnp.cos(ang))
    return pe.astype(jnp.float32)


@functools.partial(
    pl.kernel,
    out_type=jax.ShapeDtypeStruct((TOTAL, DMODEL), jnp.float32),
    mesh=plsc.VectorSubcoreMesh(core_axis_name="c", subcore_axis_name="s"),
    scratch_types=[
        pltpu.VMEM((BATCH * L_PER_W,), jnp.int32),
        pltpu.VMEM((L_PER_W, DMODEL), jnp.float32),
    ]
    + [pltpu.VMEM((CHUNK, DMODEL), jnp.float32) for _ in range(NBUF)]
    + [pltpu.SemaphoreType.DMA for _ in range(2 * NBUF + 1)],
)
def _sc_embed(ids_hbm, pe_hbm, table_hbm, out_hbm, idx_v, pe_v, *rest):
    bufs = rest[:NBUF]
    gsems = rest[NBUF:2 * NBUF]
    osems = rest[2 * NBUF:3 * NBUF]
    pe_sem = rest[3 * NBUF]

    wid = lax.axis_index("s") * 2 + lax.axis_index("c")
    l0 = wid * L_PER_W

    for b in range(BATCH):
        pltpu.sync_copy(
            ids_hbm.at[pl.ds(b * SEQLEN + l0, L_PER_W)],
            idx_v.at[pl.ds(b * L_PER_W, L_PER_W)],
        )
    pe_cp = pltpu.async_copy(pe_hbm.at[pl.ds(l0, L_PER_W)], pe_v, pe_sem)

    def gather(c):
        b, m = c // M_PER_B, c % M_PER_B
        return pltpu.async_copy(
            table_hbm.at[idx_v.at[pl.ds((b * L_PER_W + m * CHUNK), CHUNK)]],
            bufs[c % NBUF], gsems[c % NBUF],
        )

    gh = [None] * NBUF
    oh = [None] * NBUF
    for c in range(NBUF):
        gh[c] = gather(c)
    pe_cp.wait()

    for c in range(NCHUNKS):
        i = c % NBUF
        b, m = c // M_PER_B, c % M_PER_B
        gh[i].wait()

        @plsc.parallel_loop(0, CHUNK * NVEC, unroll=8)
        def add_pe(k, _buf=bufs[i], _m=m):
            r = k >> 6
            off = (k & 63) * 16
            plsc.addupdate(
                _buf.at[r, pl.ds(off, 16)],
                pe_v[_m * CHUNK + r, pl.ds(off, 16)],
            )

        nxt = c - 1 + NBUF
        if c >= 1 and nxt < NCHUNKS:
            j = (c - 1) % NBUF
            oh[j].wait()
            gh[j] = gather(nxt)

        out_base = b * SEQLEN + l0 + m * CHUNK
        oh[i] = pltpu.async_copy(bufs[i], out_hbm.at[pl.ds(out_base, CHUNK)],
                                 osems[i])
    for i in range(NBUF):
        oh[i].wait()


def kernel(seq, table):
    pe = _position_embedding()
    flat_ids = seq.reshape(TOTAL).astype(jnp.int32)
    out = _sc_embed(flat_ids, pe, table)
    return out.reshape(BATCH, SEQLEN, DMODEL)
